# baseline (speedup 1.0000x reference)
import functools

import jax
import jax.numpy as jnp
from jax import lax
from jax.experimental import pallas as pl
from jax.experimental.pallas import tpu as pltpu
from jax.experimental.pallas import tpu_sc as plsc

_NUM_CORES = 2
_NUM_SUBCORES = 16
_NUM_WORKERS = _NUM_CORES * _NUM_SUBCORES
_LANES = 16

_N = 200000
_C = 64
_B = 4
_H = 256
_W = 256
_CT = 128
_NZERO = 2048
_ZERO_ROW = _B * _H * _W
_TROWS = _B * _H * _W + _NZERO

_HB = 8
_TBLK = _HB * _W
_NREAL = (_B * _H) // _HB
_TGRID = _NREAL + _NZERO // _TBLK

_K = 400
_NCHUNKS = _N // _K
_CHUNKS_PER_WORKER = -(-_NCHUNKS // _NUM_WORKERS)
_GSUB = 80
_NGATHER = _K // _GSUB
_VPERROW = _C // _LANES


def _table_index_map(g):
    gg = jnp.minimum(g, _NREAL - 1)
    return (gg // (_H // _HB), 0, gg % (_H // _HB), 0)


@functools.partial(
    pl.pallas_call,
    out_shape=jax.ShapeDtypeStruct((_TROWS, _CT), jnp.float32),
    grid=(_TGRID,),
    in_specs=[pl.BlockSpec((1, _C, _HB, _W), _table_index_map)],
    out_specs=pl.BlockSpec((_TBLK, _CT), lambda g: (g, 0)),
)
def _build_table(in_ref, out_ref):
    g = pl.program_id(0)

    @pl.when(g < _NREAL)
    def _real():
        x = in_ref[0].reshape(_C, _TBLK)
        out_ref[...] = jnp.concatenate(
            [x.T, jnp.zeros((_TBLK, _CT - _C), jnp.float32)], axis=1)

    @pl.when(g >= _NREAL)
    def _zeros():
        out_ref[...] = jnp.zeros((_TBLK, _CT), jnp.float32)


def _make_sc_kernel():
    mesh = plsc.VectorSubcoreMesh(core_axis_name="c", subcore_axis_name="s")

    @functools.partial(
        pl.kernel,
        out_type=jax.ShapeDtypeStruct((_N, _C), jnp.float32),
        mesh=mesh,
        scratch_types=[
            pltpu.VMEM((_K, _C), jnp.float32),
            pltpu.VMEM((_K, _CT), jnp.float32),
            [pltpu.VMEM((_K,), jnp.int32)] * 8,
            [pltpu.VMEM((_NGATHER, _GSUB), jnp.int32)] * 2,
            pltpu.SemaphoreType.DMA,
            pltpu.SemaphoreType.DMA,
            pltpu.SemaphoreType.DMA,
            pltpu.SemaphoreType.DMA,
        ],
    )
    def sc_kernel(vf_hbm, table_hbm, gx_hbm, gy_hbm, b_hbm, m_hbm, out_hbm,
                  v_v, g_v, cbufs, r_vs,
                  vf_sem, c_sem, g_sem, wb_sem):
        wid = lax.axis_index("s") * _NUM_CORES + lax.axis_index("c")

        def chunk_descs(t):
            p = t % 2
            cid = wid + t * _NUM_WORKERS
            base = cid * _K
            rows = pl.ds(base, _K)
            r_v = r_vs[p]
            c_ds = [
                pltpu.make_async_copy(src.at[rows], cbufs[4 * p + i], c_sem)
                for i, src in enumerate((gx_hbm, gy_hbm, b_hbm, m_hbm))
            ]
            vf_d = pltpu.make_async_copy(vf_hbm.at[rows, :], v_v, vf_sem)
            g_ds = [
                pltpu.make_async_copy(
                    table_hbm.at[r_v.at[j]],
                    g_v.at[pl.ds(j * _GSUB, _GSUB), :],
                    g_sem,
                ) for j in range(_NGATHER)
            ]
            wb_ds = [
                pltpu.make_async_copy(
                    v_v.at[pl.ds(j * _GSUB, _GSUB), :],
                    out_hbm.at[pl.ds(base + j * _GSUB, _GSUB), :],
                    wb_sem,
                ) for j in range(_NGATHER)
            ]
            return cid, c_ds, vf_d, g_ds, wb_ds

        def rbody_for(t):
            p = t % 2
            gx_v, gy_v, b_v, m_v = cbufs[4 * p:4 * p + 4]
            r_v = r_vs[p]

            @plsc.parallel_loop(0, _K // _LANES, unroll=2)
            def rbody(j):
                sl = pl.ds(j * _LANES, _LANES)
                x16 = gx_v[sl]
                y16 = gy_v[sl]
                b16 = b_v[sl]
                m16 = m_v[sl]
                r16 = b16 * (_H * _W) + y16 * _W + x16
                zero16 = (_ZERO_ROW
                          + ((j * _LANES + lax.iota(jnp.int32, _LANES))
                             & (_NZERO - 1)))
                r16 = jnp.where(m16 != 0, r16, zero16)
                nvr = _GSUB // _LANES
                r_v[j // nvr, pl.ds((j % nvr) * _LANES, _LANES)] = r16

        descs = [chunk_descs(t) for t in range(_CHUNKS_PER_WORKER)]

        cid0, c_ds0, vf_d0, g_ds0, _ = descs[0]

        @pl.when(cid0 < _NCHUNKS)
        def _prologue():
            for d in c_ds0:
                d.start()
            vf_d0.start()
            for d in c_ds0:
                d.wait()
            rbody_for(0)
            for d in g_ds0:
                d.start()

        for t in range(_CHUNKS_PER_WORKER):
            cid, _, vf_d, g_ds, wb_ds = descs[t]
            nxt = descs[t + 1] if t + 1 < _CHUNKS_PER_WORKER else None

            if nxt is not None:
                cid1, c_ds1, vf_d1, g_ds1, _ = nxt

                @pl.when(cid1 < _NCHUNKS)
                def _coords_next():
                    for d in c_ds1:
                        d.start()

            @pl.when(cid < _NCHUNKS)
            def _vf_wait():
                vf_d.wait()

            if nxt is not None:
                @pl.when(cid1 < _NCHUNKS)
                def _rbody_next():
                    for d in c_ds1:
                        d.wait()
                    rbody_for(t + 1)

            for j in range(_NGATHER):
                @pl.when(cid < _NCHUNKS)
                def _sub():
                    g_ds[j].wait()

                    @plsc.parallel_loop(j * _GSUB, (j + 1) * _GSUB, unroll=4)
                    def fbody(i):
                        for s in range(_VPERROW):
                            sl = pl.ds(s * _LANES, _LANES)
                            plsc.addupdate(v_v.at[i, sl], g_v[i, sl])

                    wb_ds[j].start()

                if nxt is not None:
                    @pl.when(cid1 < _NCHUNKS)
                    def _gather_next():
                        g_ds1[j].start()

            if nxt is not None:
                @pl.when(cid1 < _NCHUNKS)
                def _advance():
                    for d in wb_ds:
                        d.wait()
                    vf_d1.start()

        for d in descs[-1][4]:
            d.wait()

    return sc_kernel


_sc_kernel = _make_sc_kernel()


def kernel(voxel_features, image_feat, image_grid, batch_idx, point_mask):
    table = _build_table(image_feat)
    gx = image_grid[:, 0].astype(jnp.int32)
    gy = image_grid[:, 1].astype(jnp.int32)
    bi = batch_idx.astype(jnp.int32)
    m = point_mask.astype(jnp.int32)
    return _sc_kernel(voxel_features, table, gx, gy, bi, m)


# speedup vs baseline: 1.0577x; 1.0577x over previous
import functools

import jax
import jax.numpy as jnp
from jax import lax
from jax.experimental import pallas as pl
from jax.experimental.pallas import tpu as pltpu
from jax.experimental.pallas import tpu_sc as plsc

_NUM_CORES = 2
_NUM_SUBCORES = 16
_NUM_WORKERS = _NUM_CORES * _NUM_SUBCORES
_LANES = 16

_N = 200000
_C = 64
_B = 4
_H = 256
_W = 256
_CT = 128
_NZERO = 2048
_ZERO_ROW = _B * _H * _W

_K = 400
_NCHUNKS = _N // _K
_CHUNKS_PER_WORKER = -(-_NCHUNKS // _NUM_WORKERS)
_GSUB = 80
_NGATHER = _K // _GSUB
_VPERROW = _C // _LANES


def _make_sc_kernel():
    mesh = plsc.VectorSubcoreMesh(core_axis_name="c", subcore_axis_name="s")

    @functools.partial(
        pl.kernel,
        out_type=jax.ShapeDtypeStruct((_N, _C), jnp.float32),
        mesh=mesh,
        scratch_types=[
            pltpu.VMEM((_K, _C), jnp.float32),
            pltpu.VMEM((_K, _CT), jnp.float32),
            pltpu.VMEM((_K,), jnp.int32),
            pltpu.VMEM((_K,), jnp.int32),
            pltpu.VMEM((_K,), jnp.int32),
            pltpu.VMEM((_K,), jnp.int32),
            pltpu.VMEM((_NGATHER, _GSUB), jnp.int32),
            pltpu.SemaphoreType.DMA,
            pltpu.SemaphoreType.DMA,
            pltpu.SemaphoreType.DMA,
            pltpu.SemaphoreType.DMA,
        ],
    )
    def sc_kernel(vf_hbm, table_hbm, gx_hbm, gy_hbm, b_hbm, m_hbm, out_hbm,
                  v_v, g_v, gx_v, gy_v, b_v, m_v, r_v,
                  vf_sem, c_sem, g_sem, wb_sem):
        wid = lax.axis_index("s") * _NUM_CORES + lax.axis_index("c")

        pending_wb = []

        for t in range(_CHUNKS_PER_WORKER):
            cid = wid + t * _NUM_WORKERS
            base = cid * _K
            rows = pl.ds(base, _K)
            vf_d = pltpu.make_async_copy(vf_hbm.at[rows, :], v_v, vf_sem)
            c_ds = [
                pltpu.make_async_copy(src.at[rows], dst, c_sem)
                for src, dst in ((gx_hbm, gx_v), (gy_hbm, gy_v),
                                 (b_hbm, b_v), (m_hbm, m_v))
            ]
            g_ds = [
                pltpu.make_async_copy(
                    table_hbm.at[r_v.at[j]],
                    g_v.at[pl.ds(j * _GSUB, _GSUB), :],
                    g_sem,
                ) for j in range(_NGATHER)
            ]
            wb_ds = [
                pltpu.make_async_copy(
                    v_v.at[pl.ds(j * _GSUB, _GSUB), :],
                    out_hbm.at[pl.ds(base + j * _GSUB, _GSUB), :],
                    wb_sem,
                ) for j in range(_NGATHER)
            ]

            @pl.when(cid < _NCHUNKS)
            def _chunk():
                for d in pending_wb:
                    d.wait()

                vf_d.start()
                for d in c_ds:
                    d.start()
                for d in c_ds:
                    d.wait()

                @plsc.parallel_loop(0, _K // _LANES, unroll=2)
                def rbody(j):
                    sl = pl.ds(j * _LANES, _LANES)
                    x16 = gx_v[sl]
                    y16 = gy_v[sl]
                    b16 = b_v[sl]
                    m16 = m_v[sl]
                    r16 = b16 * (_H * _W) + y16 * _W + x16
                    zero16 = (_ZERO_ROW
                              + ((j * _LANES + lax.iota(jnp.int32, _LANES))
                                 & (_NZERO - 1)))
                    r16 = jnp.where(m16 != 0, r16, zero16)
                    nvr = _GSUB // _LANES
                    r_v[j // nvr, pl.ds((j % nvr) * _LANES, _LANES)] = r16

                for d in g_ds:
                    d.start()
                vf_d.wait()

                for j in range(_NGATHER):
                    g_ds[j].wait()

                    @plsc.parallel_loop(j * _GSUB, (j + 1) * _GSUB, unroll=4)
                    def fbody(i):
                        for s in range(_VPERROW):
                            sl = pl.ds(s * _LANES, _LANES)
                            plsc.addupdate(v_v.at[i, sl], g_v[i, sl])

                    wb_ds[j].start()

            pending_wb = wb_ds

        for d in pending_wb:
            d.wait()

    return sc_kernel


_sc_kernel = _make_sc_kernel()


def kernel(voxel_features, image_feat, image_grid, batch_idx, point_mask):
    feats = jnp.transpose(image_feat, (0, 2, 3, 1)).reshape(_B * _H * _W, _C)
    table = jnp.zeros((_B * _H * _W + _NZERO, _CT), jnp.float32)
    table = lax.dynamic_update_slice(table, feats, (0, 0))
    gx = image_grid[:, 0].astype(jnp.int32)
    gy = image_grid[:, 1].astype(jnp.int32)
    bi = batch_idx.astype(jnp.int32)
    m = point_mask.astype(jnp.int32)
    return _sc_kernel(voxel_features, table, gx, gy, bi, m)
